# baseline (speedup 1.0000x reference)
import functools

import jax
import jax.numpy as jnp
from jax.experimental import pallas as pl
from jax.experimental.pallas import tpu as pltpu

_COEFF = 0.01
_BIG_I = 2**30


def _router_body(x_ref, wg_ref, we_ref, aux_ref):
    xf = x_ref[...]
    wg = wg_ref[...]
    T, _ = xf.shape
    E = wg.shape[0]
    logits = jnp.dot(xf, wg.T, preferred_element_type=jnp.float32)
    iota = jax.lax.broadcasted_iota(jnp.int32, logits.shape, 1)
    m1 = jnp.max(logits, axis=-1, keepdims=True)
    a1 = jnp.min(jnp.where(logits == m1, iota, _BIG_I), axis=-1, keepdims=True)
    oh1 = (iota == a1)
    rest = jnp.where(oh1, -jnp.inf, logits)
    m2 = jnp.max(rest, axis=-1, keepdims=True)
    a2 = jnp.min(jnp.where(rest == m2, iota, _BIG_I), axis=-1, keepdims=True)
    oh2 = (iota == a2)
    e2 = jnp.exp(m2 - m1)
    denom = 1.0 + e2
    w1 = 1.0 / denom
    w2 = e2 / denom
    we = jnp.where(oh1, w1, 0.0) + jnp.where(oh2, w2, 0.0)
    we_ref[...] = jnp.concatenate([we, jnp.ones((T, 1), jnp.float32)], axis=1)
    gates = jnp.exp(logits - m1)
    gates = gates / jnp.sum(gates, axis=-1, keepdims=True)
    f = jnp.mean(oh1.astype(jnp.float32), axis=0)
    P = jnp.mean(gates, axis=0)
    aux_ref[0, 0] = _COEFF * E * jnp.sum(f * P)


def _ffn_body(we_ref, x_ref, gw_ref, uw_ref, dw_ref, out_ref):
    e = pl.program_id(0)
    h = pl.program_id(1)

    @pl.when((e == 0) & (h == 0))
    def _():
        out_ref[...] = jnp.zeros_like(out_ref)

    xf = x_ref[...]
    g = jnp.dot(xf, gw_ref[0].T, preferred_element_type=jnp.float32)
    u = jnp.dot(xf, uw_ref[0].T, preferred_element_type=jnp.float32)
    gelu_g = 0.5 * g * (1.0 + jax.lax.erf(g * (2.0 ** -0.5)))
    hpart = gelu_g * u
    y = jnp.dot(hpart, dw_ref[0], preferred_element_type=jnp.float32)
    EW = we_ref.shape[1]
    oh = (jax.lax.broadcasted_iota(jnp.int32, (EW, 1), 0) == e).astype(jnp.float32)
    we_col = jnp.dot(we_ref[...], oh, preferred_element_type=jnp.float32)
    out_ref[...] += we_col * y


def kernel(x, Wg, gate_w, up_w, down_w, sh_gate, sh_up, sh_down):
    B, T, D = x.shape
    E, H, _ = gate_w.shape
    xf = x.reshape(B * T, D)
    TT = B * T

    we, aux = pl.pallas_call(
        _router_body,
        out_shape=(
            jax.ShapeDtypeStruct((TT, E + 1), jnp.float32),
            jax.ShapeDtypeStruct((1, 1), jnp.float32),
        ),
        out_specs=(
            pl.BlockSpec(memory_space=pltpu.VMEM),
            pl.BlockSpec(memory_space=pltpu.SMEM),
        ),
    )(xf, Wg)

    gw_all = jnp.concatenate([gate_w, sh_gate[None]], axis=0)
    uw_all = jnp.concatenate([up_w, sh_up[None]], axis=0)
    dw_all = jnp.concatenate([down_w, sh_down[None]], axis=0)
    dw_all = jnp.swapaxes(dw_all, 1, 2)

    HT = 128
    NH = H // HT
    out = pl.pallas_call(
        _ffn_body,
        grid=(E + 1, NH),
        in_specs=[
            pl.BlockSpec((TT, E + 1), lambda e, h: (0, 0)),
            pl.BlockSpec((TT, D), lambda e, h: (0, 0)),
            pl.BlockSpec((1, HT, D), lambda e, h: (e, h, 0)),
            pl.BlockSpec((1, HT, D), lambda e, h: (e, h, 0)),
            pl.BlockSpec((1, HT, D), lambda e, h: (e, h, 0)),
        ],
        out_specs=pl.BlockSpec((TT, D), lambda e, h: (0, 0)),
        out_shape=jax.ShapeDtypeStruct((TT, D), jnp.float32),
        compiler_params=pltpu.CompilerParams(
            dimension_semantics=("arbitrary", "arbitrary"),
        ),
    )(we, xf, gw_all, uw_all, dw_all)

    return out.reshape(B, T, D), aux.reshape(())
